# baseline (speedup 1.0000x reference)
import dataclasses
import functools

import jax
import jax.numpy as jnp
from jax import lax
from jax.experimental import pallas as pl
from jax.experimental.pallas import tpu as pltpu
from jax.experimental.pallas import tpu_sc as plsc

B = 16384
L = 50
DIM = 32
NC = 2
NS = 16
NW = NC * NS
NBT = B // 128
BT_PER_W = NBT // NW
KB = 8


def _sc_gather(emb_weight, idx2d):
    @functools.partial(
        pl.kernel,
        out_type=jax.ShapeDtypeStruct((L * NBT, 128, DIM), jnp.float32),
        mesh=plsc.VectorSubcoreMesh(core_axis_name="c", subcore_axis_name="s"),
        compiler_params=dataclasses.replace(
            pltpu.CompilerParams(use_tc_tiling_on_sc=False),
            needs_layout_passes=False,
        ),
        scratch_types=[
            pltpu.VMEM((L, 128), jnp.int32),
            pltpu.VMEM((128, DIM), jnp.float32),
            pltpu.VMEM((128, DIM), jnp.float32),
            pltpu.SemaphoreType.DMA,
            pltpu.SemaphoreType.DMA,
            pltpu.SemaphoreType.DMA,
            pltpu.SemaphoreType.DMA,
        ],
    )
    def gather_kernel(table_hbm, idx_hbm, stage_hbm,
                      idxv, r0, r1, g0, g1, w0, w1):
        rows = [r0, r1]
        gs = [g0, g1]
        ws = [w0, w1]

        wid = lax.axis_index("s") * NC + lax.axis_index("c")

        @pl.loop(0, BT_PER_W)
        def _(k_bt):
            bt = wid * BT_PER_W + k_bt
            pltpu.sync_copy(idx_hbm.at[:, pl.ds(bt * 128, 128)], idxv)
            pltpu.async_copy(table_hbm.at[idxv.at[0]], rows[0], gs[0])

            def step(l, par):
                pltpu.make_async_copy(
                    table_hbm.at[idxv.at[0]], rows[par], gs[par]
                ).wait()

                @pl.when(l >= 1)
                def _():
                    pltpu.make_async_copy(
                        rows[1 - par], stage_hbm.at[0], ws[1 - par]
                    ).wait()

                @pl.when(l < L - 1)
                def _():
                    pltpu.async_copy(
                        table_hbm.at[idxv.at[l + 1]], rows[1 - par], gs[1 - par]
                    )

                pltpu.async_copy(
                    rows[par], stage_hbm.at[l * NBT + bt], ws[par]
                )

            @pl.loop(0, L, step=2)
            def _(l0):
                step(l0, 0)
                step(l0 + 1, 1)

            pltpu.make_async_copy(rows[1], stage_hbm.at[0], ws[1]).wait()

    return gather_kernel(emb_weight, idx2d)


def _tc_transpose(stage2d):
    def body(in_ref, out_ref):
        for q in range(KB):
            sub = in_ref[pl.ds(q * DIM, DIM), :]
            subt = sub.reshape(128, DIM).T
            out_ref[0, :, q, :, :] = subt.reshape(4, 8, 128)

    return pl.pallas_call(
        body,
        out_shape=jax.ShapeDtypeStruct((L, 4, NBT, 8, 128), jnp.float32),
        grid=(L, NBT // KB),
        in_specs=[
            pl.BlockSpec((KB * DIM, 128), lambda l, k: (l * (NBT // KB) + k, 0)),
        ],
        out_specs=pl.BlockSpec(
            (1, 4, KB, 8, 128), lambda l, k: (l, 0, k, 0, 0)
        ),
        compiler_params=pltpu.CompilerParams(
            dimension_semantics=("parallel", "parallel"),
        ),
    )(stage2d)


def kernel(input, emb_weight):
    idx2d = input.T.astype(jnp.int32)
    stage = _sc_gather(emb_weight, idx2d)
    stage2d = stage.reshape(L * NBT * DIM, 128)
    out5d = _tc_transpose(stage2d)
    return out5d.transpose(2, 4, 0, 1, 3).reshape(B, L, DIM)


# speedup vs baseline: 1.2635x; 1.2635x over previous
import functools

import jax
import jax.numpy as jnp
from jax import lax
from jax.experimental import pallas as pl
from jax.experimental.pallas import tpu as pltpu
from jax.experimental.pallas import tpu_sc as plsc

B = 16384
L = 50
DIM = 32
N = B * L
NC = 2
NS = 16
NW = NC * NS
B_PER_W = B // NW
CB = 16
CHUNK = CB * L
STEPS = B_PER_W // CB
NBUF = 3


def kernel(input, emb_weight):
    idx = input.reshape(N).astype(jnp.int32)

    @functools.partial(
        pl.kernel,
        out_type=jax.ShapeDtypeStruct((B, L, DIM), jnp.float32),
        mesh=plsc.VectorSubcoreMesh(core_axis_name="c", subcore_axis_name="s"),
        compiler_params=pltpu.CompilerParams(use_tc_tiling_on_sc=False),
        scratch_types=(
            [pltpu.VMEM((CHUNK,), jnp.int32) for _ in range(NBUF)]
            + [pltpu.VMEM((CHUNK, DIM), jnp.float32) for _ in range(NBUF)]
            + [pltpu.SemaphoreType.DMA for _ in range(4)]
        ),
    )
    def gather_kernel(table_hbm, idx_hbm, out_hbm,
                      i0, i1, i2, r0, r1, r2, g0, g1, w0, w1):
        idx_bufs = [i0, i1, i2]
        rows_bufs = [r0, r1, r2]
        gsems = [g0, g1]
        wsems = [w0, w1]

        wid = lax.axis_index("s") * NC + lax.axis_index("c")
        base_b = wid * B_PER_W

        gathers = {}

        def start_gather(c):
            b = c % NBUF
            off = (base_b + c * CB) * L
            pltpu.sync_copy(idx_hbm.at[pl.ds(off, CHUNK)], idx_bufs[b])
            gathers[c] = pltpu.async_copy(
                table_hbm.at[idx_bufs[b]], rows_bufs[b], gsems[c % 2]
            )

        def fire_writes(c):
            b = c % NBUF
            b0 = base_b + c * CB

            @pl.loop(0, CB)
            def _(j):
                pltpu.async_copy(
                    rows_bufs[b].at[pl.ds(j * L, L)],
                    out_hbm.at[b0 + j],
                    wsems[c % 2],
                )

        def drain_writes(c):
            @pl.loop(0, CB)
            def _(j):
                pltpu.make_async_copy(
                    rows_bufs[c % NBUF].at[pl.ds(0, L)],
                    out_hbm.at[base_b],
                    wsems[c % 2],
                ).wait()

        start_gather(0)
        start_gather(1)
        for c in range(STEPS):
            gathers[c].wait()
            fire_writes(c)
            if c + 2 < STEPS:
                if c >= 1:
                    drain_writes(c - 1)
                start_gather(c + 2)
        drain_writes(STEPS - 2)
        drain_writes(STEPS - 1)

    out = gather_kernel(emb_weight, idx)
    return out
